# Initial kernel scaffold; baseline (speedup 1.0000x reference)
import jax
import jax.numpy as jnp
from jax.experimental import pallas as pl


def kernel(x1, x2, edges, W1, b1, Wg, bg, W3, b3, W4, b4):
    raise NotImplementedError("write your pallas kernel here")


# speedup vs baseline: 24.7400x; 24.7400x over previous
import functools

import jax
import jax.numpy as jnp
from jax import lax
from jax.experimental import pallas as pl
from jax.experimental.pallas import tpu as pltpu
from jax.experimental.pallas import tpu_sc as plsc

N = 100000
P = 102400
F = 16
E = 1600000
NW = 32
EPT = 50176
EPAD = NW * EPT
K = 8
B = 128
NCH = EPT // (K * B)
ROWS_PER_W = EPT // B
TR = P // 16
RB = 4096
GRID = P // RB

_mesh = plsc.VectorSubcoreMesh(core_axis_name="c", subcore_axis_name="s")
_sc_params = pltpu.CompilerParams(use_tc_tiling_on_sc=False)


@functools.partial(
    pl.kernel,
    out_type=jax.ShapeDtypeStruct((2, P), jnp.float32),
    mesh=_mesh,
    scratch_types=[
        pltpu.VMEM((K, B), jnp.int32),
        pltpu.VMEM((B,), jnp.float32),
        pltpu.VMEM((TR,), jnp.float32),
        pltpu.VMEM_SHARED((P,), jnp.float32),
    ],
    compiler_params=_sc_params,
)
def _deg_kernel(dst_r, out_r, didx, ones, zb, accd):
    c = lax.axis_index("c")
    s = lax.axis_index("s")
    r0 = s * TR

    def zfill(i, carry):
        zb[pl.ds(i * 16, 16)] = jnp.zeros((16,), jnp.float32)
        return carry
    lax.fori_loop(0, TR // 16, zfill, 0)
    for j in range(B // 16):
        ones[pl.ds(j * 16, 16)] = jnp.ones((16,), jnp.float32)
    pltpu.sync_copy(zb, accd.at[pl.ds(r0, TR)])
    plsc.subcore_barrier()

    w = c * 16 + s
    row0 = w * ROWS_PER_W

    def chunk(i, carry):
        pltpu.sync_copy(dst_r.at[pl.ds(row0 + i * K, K)], didx)
        for j in range(K):
            pltpu.sync_copy(ones, accd.at[didx.at[j]], add=True)
        return carry
    lax.fori_loop(0, NCH, chunk, 0)

    plsc.subcore_barrier()
    pltpu.sync_copy(accd.at[pl.ds(r0, TR)], out_r.at[c].at[pl.ds(r0, TR)])


@functools.partial(
    pl.kernel,
    out_type=jax.ShapeDtypeStruct((2, P, F), jnp.float32),
    mesh=_mesh,
    scratch_types=[
        pltpu.VMEM((K, B), jnp.int32),
        pltpu.VMEM((K, B), jnp.int32),
        pltpu.VMEM((K, B, F), jnp.float32),
        pltpu.VMEM_SHARED((P, F), jnp.float32),
        pltpu.SemaphoreType.DMA,
    ],
    compiler_params=_sc_params,
)
def _edge_kernel(src_r, dst_r, hn_r, out_r, sidx, didx, rows, acc, sem):
    c = lax.axis_index("c")
    s = lax.axis_index("s")
    r0 = s * TR
    pltpu.sync_copy(hn_r.at[pl.ds(r0, TR)], acc.at[pl.ds(r0, TR)])
    plsc.subcore_barrier()

    w = c * 16 + s
    row0 = w * ROWS_PER_W

    def chunk(i, carry):
        rbase = row0 + i * K
        pltpu.sync_copy(src_r.at[pl.ds(rbase, K)], sidx)
        pltpu.sync_copy(dst_r.at[pl.ds(rbase, K)], didx)
        cps = [pltpu.async_copy(hn_r.at[sidx.at[j]], rows.at[j], sem)
               for j in range(K)]
        for cp in cps:
            cp.wait()
        for j in range(K):
            pltpu.sync_copy(rows.at[j], acc.at[didx.at[j]], add=True)
        return carry
    lax.fori_loop(0, NCH, chunk, 0)

    plsc.subcore_barrier()
    pltpu.sync_copy(acc.at[pl.ds(r0, TR)], out_r.at[c].at[pl.ds(r0, TR)])


def _tcA_body(xin_ref, W4_ref, b4_ref, WgB_ref, W1_ref, b1_ref, WgA_ref,
              c_ref, h0_ref, cs_ref):
    i = pl.program_id(0)
    x = xin_ref[...]
    bf = jnp.dot(x, W4_ref[...], preferred_element_type=jnp.float32) + b4_ref[...]
    rows = lax.broadcasted_iota(jnp.int32, (RB, 1), 0) + i * RB
    m = rows < N
    bf = jnp.where(m, bf, 0.0)
    cc = jnp.dot(bf, WgB_ref[...], preferred_element_type=jnp.float32)
    c_ref[...] = cc
    a0 = jnp.maximum(
        jnp.dot(x, W1_ref[...], preferred_element_type=jnp.float32) + b1_ref[...], 0.0)
    a0 = jnp.where(m, a0, 0.0)
    h0_ref[...] = jnp.dot(a0, WgA_ref[...], preferred_element_type=jnp.float32) + cc

    @pl.when(i == 0)
    def _():
        cs_ref[...] = jnp.zeros_like(cs_ref)
    cs_ref[...] += jnp.sum(bf, axis=0, keepdims=True)


def _tcB_body(deg_ref, h0_ref, dinv_ref, hn_ref):
    d = deg_ref[...]
    dinv = lax.rsqrt(1.0 + d[0] + d[1])
    dinv16 = dinv[:, None] * jnp.ones((1, F), jnp.float32)
    dinv_ref[...] = dinv16
    hn_ref[...] = h0_ref[...] * dinv16


def _tcL_body(a0_ref, a1_ref, hn_ref, dinv_ref, c_ref, WgA_ref, bg_ref, out_ref):
    i = pl.program_id(0)
    hn = hn_ref[...]
    sacc = a0_ref[0] + a1_ref[0] - hn
    dinv = dinv_ref[...]
    a = jnp.maximum(dinv * sacc + bg_ref[...], 0.0)
    rows = lax.broadcasted_iota(jnp.int32, (RB, 1), 0) + i * RB
    a = jnp.where(rows < N, a, 0.0)
    out_ref[...] = (jnp.dot(a, WgA_ref[...],
                            preferred_element_type=jnp.float32) + c_ref[...]) * dinv


def _tcF_body(a0_ref, a1_ref, hn_ref, dinv_ref, bg_ref, csbf_ref,
              w3a_ref, w3b_ref, b3_ref, cs_ref, res_ref):
    i = pl.program_id(0)
    hn = hn_ref[...]
    sacc = a0_ref[0] + a1_ref[0] - hn
    a = jnp.maximum(dinv_ref[...] * sacc + bg_ref[...], 0.0)
    rows = lax.broadcasted_iota(jnp.int32, (RB, 1), 0) + i * RB
    a = jnp.where(rows < N, a, 0.0)

    @pl.when(i == 0)
    def _():
        cs_ref[...] = jnp.zeros_like(cs_ref)
    cs_ref[...] += jnp.sum(a, axis=0, keepdims=True)

    @pl.when(i == pl.num_programs(0) - 1)
    def _():
        t = (jnp.sum(cs_ref[...] * w3a_ref[...])
             + jnp.sum(csbf_ref[...] * w3b_ref[...])) / jnp.float32(N) + b3_ref[0, 0]
        res_ref[...] = jnp.tanh(t) * jnp.ones((1, 1), jnp.float32)


def _row_spec(width):
    return pl.BlockSpec((RB, width), lambda i: (i, 0))


def _const_spec(shape):
    return pl.BlockSpec(shape, lambda i: tuple(0 for _ in shape))


_tcA = pl.pallas_call(
    _tcA_body,
    grid=(GRID,),
    in_specs=[_row_spec(32), _const_spec((32, 32)), _const_spec((1, 32)),
              _const_spec((32, F)), _const_spec((32, F)), _const_spec((1, F)),
              _const_spec((F, F))],
    out_specs=[_row_spec(F), _row_spec(F), _const_spec((1, 32))],
    out_shape=[jax.ShapeDtypeStruct((P, F), jnp.float32),
               jax.ShapeDtypeStruct((P, F), jnp.float32),
               jax.ShapeDtypeStruct((1, 32), jnp.float32)],
)

_tcB = pl.pallas_call(
    _tcB_body,
    grid=(GRID,),
    in_specs=[pl.BlockSpec((2, RB), lambda i: (0, i)), _row_spec(F)],
    out_specs=[_row_spec(F), _row_spec(F)],
    out_shape=[jax.ShapeDtypeStruct((P, F), jnp.float32),
               jax.ShapeDtypeStruct((P, F), jnp.float32)],
)

_acc_spec = pl.BlockSpec((1, RB, F), lambda i: (0, i, 0))
_acc1_spec = pl.BlockSpec((1, RB, F), lambda i: (1, i, 0))

_tcL = pl.pallas_call(
    _tcL_body,
    grid=(GRID,),
    in_specs=[_acc_spec, _acc1_spec, _row_spec(F), _row_spec(F), _row_spec(F),
              _const_spec((F, F)), _const_spec((1, F))],
    out_specs=_row_spec(F),
    out_shape=jax.ShapeDtypeStruct((P, F), jnp.float32),
)

_tcF = pl.pallas_call(
    _tcF_body,
    grid=(GRID,),
    in_specs=[_acc_spec, _acc1_spec, _row_spec(F), _row_spec(F),
              _const_spec((1, F)), _const_spec((1, 32)),
              _const_spec((1, F)), _const_spec((1, 32)), _const_spec((1, 1))],
    out_specs=[_const_spec((1, F)), _const_spec((1, 1))],
    out_shape=[jax.ShapeDtypeStruct((1, F), jnp.float32),
               jax.ShapeDtypeStruct((1, 1), jnp.float32)],
)


def kernel(x1, x2, edges, W1, b1, Wg, bg, W3, b3, W4, b4):
    f32 = jnp.float32
    xin = jnp.zeros((P, 32), f32)
    xin = xin.at[:N, :15].set(x1)
    xin = xin.at[:N, 15:19].set(jnp.tile(x2, (20, 1)))
    W4p = jnp.zeros((32, 32), f32).at[:19, :19].set(W4)
    b4p = jnp.zeros((1, 32), f32).at[0, :19].set(b4)
    WgBp = jnp.zeros((32, F), f32).at[:19, :15].set(Wg[15:])
    W1p = jnp.zeros((32, F), f32).at[:15, :15].set(W1)
    b1p = jnp.zeros((1, F), f32).at[0, :15].set(b1)
    WgAp = jnp.zeros((F, F), f32).at[:15, :15].set(Wg[:15])
    bg16 = jnp.zeros((1, F), f32).at[0, :15].set(bg)
    w3a = jnp.zeros((1, F), f32).at[0, :15].set(W3[:15, 0])
    w3b = jnp.zeros((1, 32), f32).at[0, :19].set(W3[15:, 0])
    b3s = jnp.reshape(b3, (1, 1)).astype(f32)

    npad = EPAD - E
    padi = (N + (jnp.arange(npad, dtype=jnp.int32) % 64))
    src2d = jnp.concatenate([edges[0], padi]).reshape(EPAD // B, B)
    dst2d = jnp.concatenate([edges[1], padi]).reshape(EPAD // B, B)

    deg = _deg_kernel(dst2d)
    c_arr, h0, csum_bf = _tcA(xin, W4p, b4p, WgBp, W1p, b1p, WgAp)
    dinv16, hn = _tcB(deg, h0)
    for k in range(5):
        acc = _edge_kernel(src2d, dst2d, hn)
        if k < 4:
            hn = _tcL(acc, acc, hn, dinv16, c_arr, WgAp, bg16)
        else:
            _, res = _tcF(acc, acc, hn, dinv16, bg16, csum_bf, w3a, w3b, b3s)
    return jnp.reshape(res, ())


# speedup vs baseline: 26.5072x; 1.0714x over previous
import functools

import jax
import jax.numpy as jnp
from jax import lax
from jax.experimental import pallas as pl
from jax.experimental.pallas import tpu as pltpu
from jax.experimental.pallas import tpu_sc as plsc

N = 100000
P = 102400
F = 16
BLK = 4096
GRID = P // BLK
E = 1600000
NW = 32
EPT = 50176
EPAD = NW * EPT
K = 8
B = 128
NCH = EPT // (K * B)
ROWS_PER_W = EPT // B
TR = P // 16

_mesh = plsc.VectorSubcoreMesh(core_axis_name="c", subcore_axis_name="s")
_sc_params = pltpu.CompilerParams(use_tc_tiling_on_sc=False)


@functools.partial(
    pl.kernel,
    out_type=jax.ShapeDtypeStruct((2, P), jnp.float32),
    mesh=_mesh,
    scratch_types=[
        pltpu.VMEM((K, B), jnp.int32),
        pltpu.VMEM((B,), jnp.float32),
        pltpu.VMEM((TR,), jnp.float32),
        pltpu.VMEM_SHARED((P,), jnp.float32),
    ],
    compiler_params=_sc_params,
)
def _deg_kernel(dst_r, out_r, didx, ones, zb, accd):
    c = lax.axis_index("c")
    s = lax.axis_index("s")
    r0 = s * TR

    def zfill(i, carry):
        zb[pl.ds(i * 16, 16)] = jnp.zeros((16,), jnp.float32)
        return carry
    lax.fori_loop(0, TR // 16, zfill, 0)
    for j in range(B // 16):
        ones[pl.ds(j * 16, 16)] = jnp.ones((16,), jnp.float32)
    pltpu.sync_copy(zb, accd.at[pl.ds(r0, TR)])
    plsc.subcore_barrier()

    w = c * 16 + s
    row0 = w * ROWS_PER_W

    def chunk(i, carry):
        pltpu.sync_copy(dst_r.at[pl.ds(row0 + i * K, K)], didx)
        for j in range(K):
            pltpu.sync_copy(ones, accd.at[didx.at[j]], add=True)
        return carry
    lax.fori_loop(0, NCH, chunk, 0)

    plsc.subcore_barrier()
    pltpu.sync_copy(accd.at[pl.ds(r0, TR)], out_r.at[c].at[pl.ds(r0, TR)])


@functools.partial(
    pl.kernel,
    out_type=jax.ShapeDtypeStruct((2, P, F), jnp.float32),
    mesh=_mesh,
    scratch_types=[
        pltpu.VMEM((K, B), jnp.int32),
        pltpu.VMEM((K, B), jnp.int32),
        pltpu.VMEM((K, B, F), jnp.float32),
        pltpu.VMEM_SHARED((P, F), jnp.float32),
        pltpu.SemaphoreType.DMA,
    ],
    compiler_params=_sc_params,
)
def _edge_kernel(src_r, dst_r, hn_r, out_r, sidx, didx, rows, acc, sem):
    c = lax.axis_index("c")
    s = lax.axis_index("s")
    r0 = s * TR
    pltpu.sync_copy(hn_r.at[pl.ds(r0, TR)], acc.at[pl.ds(r0, TR)])
    plsc.subcore_barrier()

    w = c * 16 + s
    row0 = w * ROWS_PER_W

    def chunk(i, carry):
        rbase = row0 + i * K
        pltpu.sync_copy(src_r.at[pl.ds(rbase, K)], sidx)
        pltpu.sync_copy(dst_r.at[pl.ds(rbase, K)], didx)
        cps = [pltpu.async_copy(hn_r.at[sidx.at[j]], rows.at[j], sem)
               for j in range(K)]
        for cp in cps:
            cp.wait()
        for j in range(K):
            pltpu.sync_copy(rows.at[j], acc.at[didx.at[j]], add=True)
        return carry
    lax.fori_loop(0, NCH, chunk, 0)

    plsc.subcore_barrier()
    pltpu.sync_copy(acc.at[pl.ds(r0, TR)], out_r.at[c].at[pl.ds(r0, TR)])


def _node_mask(i):
    rows = lax.broadcasted_iota(jnp.int32, (BLK, F), 0)
    return (i * BLK + rows) < N


def _tcA_body(x1_ref, x2_ref, W4a_ref, W4b_ref, b4_ref, WgB_ref,
              W1_ref, b1_ref, WgA_ref, c_ref, h0_ref, cs_ref):
    i = pl.program_id(0)
    x1b = x1_ref[...]
    x2b = x2_ref[...]
    bf = (jnp.dot(x1b, W4a_ref[...], preferred_element_type=jnp.float32)
          + jnp.dot(x2b, W4b_ref[...], preferred_element_type=jnp.float32)
          + b4_ref[...])
    cc = jnp.dot(bf, WgB_ref[...], preferred_element_type=jnp.float32)
    c_ref[...] = cc
    a0 = jnp.maximum(
        jnp.dot(x1b, W1_ref[...], preferred_element_type=jnp.float32)
        + b1_ref[...], 0.0)
    h0_ref[...] = jnp.dot(a0, WgA_ref[...],
                          preferred_element_type=jnp.float32) + cc

    @pl.when(i == 0)
    def _():
        cs_ref[...] = jnp.zeros_like(cs_ref)
    cs_ref[...] += jnp.sum(bf, axis=0, keepdims=True)


def _tcB_body(deg_ref, h0_ref, dinv_ref, hn_ref):
    i = pl.program_id(0)
    d = deg_ref[...]
    dn = lax.rsqrt(1.0 + d[0] + d[1])
    dinv_ref[...] = dn
    hn_ref[...] = jnp.where(_node_mask(i), h0_ref[...] * dn, 0.0)


def _tcL_body(a0_ref, a1_ref, hn_ref, dinv_ref, c_ref, WgA_ref, bg_ref,
              out_ref):
    i = pl.program_id(0)
    hn = hn_ref[...]
    sacc = a0_ref[0] + a1_ref[0] - hn
    dinv = dinv_ref[...]
    m = _node_mask(i)
    a = jnp.where(m, jnp.maximum(dinv * sacc + bg_ref[...], 0.0), 0.0)
    h = jnp.dot(a, WgA_ref[...], preferred_element_type=jnp.float32)
    out_ref[...] = jnp.where(m, (h + c_ref[...]) * dinv, 0.0)


def _tcF_body(a0_ref, a1_ref, hn_ref, dinv_ref, bg_ref, csbf_ref,
              w3a_ref, w3b_ref, b3_ref, cs_ref, res_ref):
    i = pl.program_id(0)
    hn = hn_ref[...]
    sacc = a0_ref[0] + a1_ref[0] - hn
    a = jnp.where(_node_mask(i),
                  jnp.maximum(dinv_ref[...] * sacc + bg_ref[...], 0.0), 0.0)

    @pl.when(i == 0)
    def _():
        cs_ref[...] = jnp.zeros_like(cs_ref)
    cs_ref[...] += jnp.sum(a, axis=0, keepdims=True)

    @pl.when(i == pl.num_programs(0) - 1)
    def _():
        t = (jnp.sum(cs_ref[...] * w3a_ref[...])
             + jnp.sum(csbf_ref[...] * w3b_ref[...])) / jnp.float32(N) \
            + b3_ref[0, 0]
        res_ref[...] = jnp.tanh(t) * jnp.ones((1, 1), jnp.float32)


def _row_spec():
    return pl.BlockSpec((BLK, F), lambda i: (i, 0))


def _col_spec():
    return pl.BlockSpec((BLK, 1), lambda i: (i, 0))


def _const_spec(shape):
    return pl.BlockSpec(shape, lambda i: tuple(0 for _ in shape))


_tcA = pl.pallas_call(
    _tcA_body,
    grid=(20,),
    in_specs=[pl.BlockSpec((5000, 15), lambda i: (i, 0)),
              _const_spec((5000, 4)),
              _const_spec((15, 32)), _const_spec((4, 32)),
              _const_spec((1, 32)), _const_spec((32, F)),
              _const_spec((15, F)), _const_spec((1, F)),
              _const_spec((F, F))],
    out_specs=[pl.BlockSpec((5000, F), lambda i: (i, 0)),
               pl.BlockSpec((5000, F), lambda i: (i, 0)),
               _const_spec((1, 32))],
    out_shape=[jax.ShapeDtypeStruct((N, F), jnp.float32),
               jax.ShapeDtypeStruct((N, F), jnp.float32),
               jax.ShapeDtypeStruct((1, 32), jnp.float32)],
)

_tcB = pl.pallas_call(
    _tcB_body,
    grid=(GRID,),
    in_specs=[pl.BlockSpec((2, BLK, 1), lambda i: (0, i, 0)), _row_spec()],
    out_specs=[_col_spec(), _row_spec()],
    out_shape=[jax.ShapeDtypeStruct((P, 1), jnp.float32),
               jax.ShapeDtypeStruct((P, F), jnp.float32)],
)

_acc0_spec = pl.BlockSpec((1, BLK, F), lambda i: (0, i, 0))
_acc1_spec = pl.BlockSpec((1, BLK, F), lambda i: (1, i, 0))

_tcL = pl.pallas_call(
    _tcL_body,
    grid=(GRID,),
    in_specs=[_acc0_spec, _acc1_spec, _row_spec(), _col_spec(), _row_spec(),
              _const_spec((F, F)), _const_spec((1, F))],
    out_specs=_row_spec(),
    out_shape=jax.ShapeDtypeStruct((P, F), jnp.float32),
)

_tcF = pl.pallas_call(
    _tcF_body,
    grid=(GRID,),
    in_specs=[_acc0_spec, _acc1_spec, _row_spec(), _col_spec(),
              _const_spec((1, F)), _const_spec((1, 32)),
              _const_spec((1, F)), _const_spec((1, 32)),
              _const_spec((1, 1))],
    out_specs=[_const_spec((1, F)), _const_spec((1, 1))],
    out_shape=[jax.ShapeDtypeStruct((1, F), jnp.float32),
               jax.ShapeDtypeStruct((1, 1), jnp.float32)],
)


def kernel(x1, x2, edges, W1, b1, Wg, bg, W3, b3, W4, b4):
    f32 = jnp.float32
    W4a = jnp.zeros((15, 32), f32).at[:, :19].set(W4[:15])
    W4b = jnp.zeros((4, 32), f32).at[:, :19].set(W4[15:])
    b4p = jnp.zeros((1, 32), f32).at[0, :19].set(b4)
    WgBp = jnp.zeros((32, F), f32).at[:19, :15].set(Wg[15:])
    W1p = jnp.zeros((15, F), f32).at[:, :15].set(W1)
    b1p = jnp.zeros((1, F), f32).at[0, :15].set(b1)
    WgAp = jnp.zeros((F, F), f32).at[:15, :15].set(Wg[:15])
    bg16 = jnp.zeros((1, F), f32).at[0, :15].set(bg)
    w3a = jnp.zeros((1, F), f32).at[0, :15].set(W3[:15, 0])
    w3b = jnp.zeros((1, 32), f32).at[0, :19].set(W3[15:, 0])
    b3s = jnp.reshape(b3, (1, 1)).astype(f32)

    npad = EPAD - E
    padi = (N + (jnp.arange(npad, dtype=jnp.int32) % 64))
    src2d = jnp.concatenate([edges[0], padi]).reshape(EPAD // B, B)
    dst2d = jnp.concatenate([edges[1], padi]).reshape(EPAD // B, B)

    deg = _deg_kernel(dst2d)
    deg3 = jnp.reshape(deg, (2, P, 1))
    c, h0, csum_bf = _tcA(x1, x2, W4a, W4b, b4p, WgBp, W1p, b1p, WgAp)
    cp = jnp.pad(c, ((0, P - N), (0, 0)))
    h0p = jnp.pad(h0, ((0, P - N), (0, 0)))
    dinv, hn = _tcB(deg3, h0p)
    for k in range(5):
        acc = _edge_kernel(src2d, dst2d, hn)
        if k < 4:
            hn = _tcL(acc, acc, hn, dinv, cp, WgAp, bg16)
        else:
            _, res = _tcF(acc, acc, hn, dinv, bg16, csum_bf, w3a, w3b, b3s)
    return jnp.reshape(res, ())


# speedup vs baseline: 30.5566x; 1.1528x over previous
import functools

import jax
import jax.numpy as jnp
from jax import lax
from jax.experimental import pallas as pl
from jax.experimental.pallas import tpu as pltpu
from jax.experimental.pallas import tpu_sc as plsc

N = 100000
P = 102400
F = 16
BLK = 4096
GRID = P // BLK
E = 1600000
NW = 32
EPT = 50688
EPAD = NW * EPT
K = 6
B = 128
NCH = EPT // (K * B)
ROWS_PER_W = EPT // B
TR = P // 16

_mesh = plsc.VectorSubcoreMesh(core_axis_name="c", subcore_axis_name="s")
_sc_params = pltpu.CompilerParams(use_tc_tiling_on_sc=False)


@functools.partial(
    pl.kernel,
    out_type=jax.ShapeDtypeStruct((2, P), jnp.float32),
    mesh=_mesh,
    scratch_types=[
        pltpu.VMEM((K, B), jnp.int32),
        pltpu.VMEM((B,), jnp.float32),
        pltpu.VMEM((TR,), jnp.float32),
        pltpu.VMEM_SHARED((P,), jnp.float32),
    ],
    compiler_params=_sc_params,
)
def _deg_kernel(dst_r, out_r, didx, ones, zb, accd):
    c = lax.axis_index("c")
    s = lax.axis_index("s")
    r0 = s * TR

    def zfill(i, carry):
        zb[pl.ds(i * 16, 16)] = jnp.zeros((16,), jnp.float32)
        return carry
    lax.fori_loop(0, TR // 16, zfill, 0)
    for j in range(B // 16):
        ones[pl.ds(j * 16, 16)] = jnp.ones((16,), jnp.float32)
    pltpu.sync_copy(zb, accd.at[pl.ds(r0, TR)])
    plsc.subcore_barrier()

    w = c * 16 + s
    row0 = w * ROWS_PER_W

    def chunk(i, carry):
        pltpu.sync_copy(dst_r.at[pl.ds(row0 + i * K, K)], didx)
        for j in range(K):
            pltpu.sync_copy(ones, accd.at[didx.at[j]], add=True)
        return carry
    lax.fori_loop(0, NCH, chunk, 0)

    plsc.subcore_barrier()
    pltpu.sync_copy(accd.at[pl.ds(r0, TR)], out_r.at[c].at[pl.ds(r0, TR)])


@functools.partial(
    pl.kernel,
    out_type=jax.ShapeDtypeStruct((2, P, F), jnp.float32),
    mesh=_mesh,
    scratch_types=[
        pltpu.VMEM((2, K, B), jnp.int32),
        pltpu.VMEM((2, K, B), jnp.int32),
        pltpu.VMEM((2, K, B, F), jnp.float32),
        pltpu.VMEM_SHARED((P, F), jnp.float32),
        pltpu.SemaphoreType.DMA,
        pltpu.SemaphoreType.DMA,
    ],
    compiler_params=_sc_params,
)
def _edge_kernel(src_r, dst_r, hn_r, out_r, sidx, didx, rows, acc,
                 sem0, sem1):
    c = lax.axis_index("c")
    s = lax.axis_index("s")
    r0 = s * TR
    pltpu.sync_copy(hn_r.at[pl.ds(r0, TR)], acc.at[pl.ds(r0, TR)])
    plsc.subcore_barrier()

    w = c * 16 + s
    row0 = w * ROWS_PER_W
    sems = (sem0, sem1)

    def load_fire(chunk, b):
        rbase = row0 + chunk * K
        pltpu.sync_copy(src_r.at[pl.ds(rbase, K)], sidx.at[b])
        pltpu.sync_copy(dst_r.at[pl.ds(rbase, K)], didx.at[b])
        for j in range(K):
            pltpu.async_copy(hn_r.at[sidx.at[b].at[j]], rows.at[b].at[j],
                             sems[b])

    def drain_scatter(b):
        for j in range(K):
            pltpu.make_async_copy(hn_r.at[sidx.at[b].at[j]],
                                  rows.at[b].at[j], sems[b]).wait()
        for j in range(K):
            pltpu.sync_copy(rows.at[b].at[j], acc.at[didx.at[b].at[j]],
                            add=True)

    load_fire(0, 0)
    load_fire(1, 1)

    def body(t, carry):
        cbase = 2 * t
        drain_scatter(0)
        load_fire(cbase + 2, 0)
        drain_scatter(1)
        load_fire(cbase + 3, 1)
        return carry
    lax.fori_loop(0, NCH // 2 - 1, body, 0)
    drain_scatter(0)
    drain_scatter(1)

    plsc.subcore_barrier()
    pltpu.sync_copy(acc.at[pl.ds(r0, TR)], out_r.at[c].at[pl.ds(r0, TR)])


def _node_mask(i):
    rows = lax.broadcasted_iota(jnp.int32, (BLK, F), 0)
    return (i * BLK + rows) < N


def _tcA_body(x1_ref, x2_ref, W4a_ref, W4b_ref, b4_ref, WgB_ref,
              W1_ref, b1_ref, WgA_ref, c_ref, h0_ref, cs_ref):
    i = pl.program_id(0)
    x1b = x1_ref[...]
    x2b = x2_ref[...]
    bf = (jnp.dot(x1b, W4a_ref[...], preferred_element_type=jnp.float32)
          + jnp.dot(x2b, W4b_ref[...], preferred_element_type=jnp.float32)
          + b4_ref[...])
    cc = jnp.dot(bf, WgB_ref[...], preferred_element_type=jnp.float32)
    c_ref[...] = cc
    a0 = jnp.maximum(
        jnp.dot(x1b, W1_ref[...], preferred_element_type=jnp.float32)
        + b1_ref[...], 0.0)
    h0_ref[...] = jnp.dot(a0, WgA_ref[...],
                          preferred_element_type=jnp.float32) + cc

    @pl.when(i == 0)
    def _():
        cs_ref[...] = jnp.zeros_like(cs_ref)
    cs_ref[...] += jnp.sum(bf, axis=0, keepdims=True)


def _tcB_body(deg_ref, h0_ref, dinv_ref, hn_ref):
    i = pl.program_id(0)
    d = deg_ref[...]
    dn = lax.rsqrt(1.0 + d[0] + d[1])
    dinv_ref[...] = dn
    hn_ref[...] = jnp.where(_node_mask(i), h0_ref[...] * dn, 0.0)


def _tcL_body(a0_ref, a1_ref, hn_ref, dinv_ref, c_ref, WgA_ref, bg_ref,
              out_ref):
    i = pl.program_id(0)
    hn = hn_ref[...]
    sacc = a0_ref[0] + a1_ref[0] - hn
    dinv = dinv_ref[...]
    m = _node_mask(i)
    a = jnp.where(m, jnp.maximum(dinv * sacc + bg_ref[...], 0.0), 0.0)
    h = jnp.dot(a, WgA_ref[...], preferred_element_type=jnp.float32)
    out_ref[...] = jnp.where(m, (h + c_ref[...]) * dinv, 0.0)


def _tcF_body(a0_ref, a1_ref, hn_ref, dinv_ref, bg_ref, csbf_ref,
              w3a_ref, w3b_ref, b3_ref, cs_ref, res_ref):
    i = pl.program_id(0)
    hn = hn_ref[...]
    sacc = a0_ref[0] + a1_ref[0] - hn
    a = jnp.where(_node_mask(i),
                  jnp.maximum(dinv_ref[...] * sacc + bg_ref[...], 0.0), 0.0)

    @pl.when(i == 0)
    def _():
        cs_ref[...] = jnp.zeros_like(cs_ref)
    cs_ref[...] += jnp.sum(a, axis=0, keepdims=True)

    @pl.when(i == pl.num_programs(0) - 1)
    def _():
        t = (jnp.sum(cs_ref[...] * w3a_ref[...])
             + jnp.sum(csbf_ref[...] * w3b_ref[...])) / jnp.float32(N) \
            + b3_ref[0, 0]
        res_ref[...] = jnp.tanh(t) * jnp.ones((1, 1), jnp.float32)


def _row_spec():
    return pl.BlockSpec((BLK, F), lambda i: (i, 0))


def _col_spec():
    return pl.BlockSpec((BLK, 1), lambda i: (i, 0))


def _const_spec(shape):
    return pl.BlockSpec(shape, lambda i: tuple(0 for _ in shape))


_tcA = pl.pallas_call(
    _tcA_body,
    grid=(20,),
    in_specs=[pl.BlockSpec((5000, 15), lambda i: (i, 0)),
              _const_spec((5000, 4)),
              _const_spec((15, 32)), _const_spec((4, 32)),
              _const_spec((1, 32)), _const_spec((32, F)),
              _const_spec((15, F)), _const_spec((1, F)),
              _const_spec((F, F))],
    out_specs=[pl.BlockSpec((5000, F), lambda i: (i, 0)),
               pl.BlockSpec((5000, F), lambda i: (i, 0)),
               _const_spec((1, 32))],
    out_shape=[jax.ShapeDtypeStruct((N, F), jnp.float32),
               jax.ShapeDtypeStruct((N, F), jnp.float32),
               jax.ShapeDtypeStruct((1, 32), jnp.float32)],
)

_tcB = pl.pallas_call(
    _tcB_body,
    grid=(GRID,),
    in_specs=[pl.BlockSpec((2, BLK, 1), lambda i: (0, i, 0)), _row_spec()],
    out_specs=[_col_spec(), _row_spec()],
    out_shape=[jax.ShapeDtypeStruct((P, 1), jnp.float32),
               jax.ShapeDtypeStruct((P, F), jnp.float32)],
)

_acc0_spec = pl.BlockSpec((1, BLK, F), lambda i: (0, i, 0))
_acc1_spec = pl.BlockSpec((1, BLK, F), lambda i: (1, i, 0))

_tcL = pl.pallas_call(
    _tcL_body,
    grid=(GRID,),
    in_specs=[_acc0_spec, _acc1_spec, _row_spec(), _col_spec(), _row_spec(),
              _const_spec((F, F)), _const_spec((1, F))],
    out_specs=_row_spec(),
    out_shape=jax.ShapeDtypeStruct((P, F), jnp.float32),
)

_tcF = pl.pallas_call(
    _tcF_body,
    grid=(GRID,),
    in_specs=[_acc0_spec, _acc1_spec, _row_spec(), _col_spec(),
              _const_spec((1, F)), _const_spec((1, 32)),
              _const_spec((1, F)), _const_spec((1, 32)),
              _const_spec((1, 1))],
    out_specs=[_const_spec((1, F)), _const_spec((1, 1))],
    out_shape=[jax.ShapeDtypeStruct((1, F), jnp.float32),
               jax.ShapeDtypeStruct((1, 1), jnp.float32)],
)


def kernel(x1, x2, edges, W1, b1, Wg, bg, W3, b3, W4, b4):
    f32 = jnp.float32
    W4a = jnp.zeros((15, 32), f32).at[:, :19].set(W4[:15])
    W4b = jnp.zeros((4, 32), f32).at[:, :19].set(W4[15:])
    b4p = jnp.zeros((1, 32), f32).at[0, :19].set(b4)
    WgBp = jnp.zeros((32, F), f32).at[:19, :15].set(Wg[15:])
    W1p = jnp.zeros((15, F), f32).at[:, :15].set(W1)
    b1p = jnp.zeros((1, F), f32).at[0, :15].set(b1)
    WgAp = jnp.zeros((F, F), f32).at[:15, :15].set(Wg[:15])
    bg16 = jnp.zeros((1, F), f32).at[0, :15].set(bg)
    w3a = jnp.zeros((1, F), f32).at[0, :15].set(W3[:15, 0])
    w3b = jnp.zeros((1, 32), f32).at[0, :19].set(W3[15:, 0])
    b3s = jnp.reshape(b3, (1, 1)).astype(f32)

    npad = EPAD - E
    padi = (N + (jnp.arange(npad, dtype=jnp.int32) % 64))
    src2d = jnp.concatenate([edges[0], padi]).reshape(EPAD // B, B)
    dst2d = jnp.concatenate([edges[1], padi]).reshape(EPAD // B, B)

    deg = _deg_kernel(dst2d)
    deg3 = jnp.reshape(deg, (2, P, 1))
    c, h0, csum_bf = _tcA(x1, x2, W4a, W4b, b4p, WgBp, W1p, b1p, WgAp)
    cp = jnp.pad(c, ((0, P - N), (0, 0)))
    h0p = jnp.pad(h0, ((0, P - N), (0, 0)))
    dinv, hn = _tcB(deg3, h0p)
    for k in range(5):
        acc = _edge_kernel(src2d, dst2d, hn)
        if k < 4:
            hn = _tcL(acc, acc, hn, dinv, cp, WgAp, bg16)
        else:
            _, res = _tcF(acc, acc, hn, dinv, bg16, csum_bf, w3a, w3b, b3s)
    return jnp.reshape(res, ())


# speedup vs baseline: 31.4741x; 1.0300x over previous
import functools

import jax
import jax.numpy as jnp
from jax import lax
from jax.experimental import pallas as pl
from jax.experimental.pallas import tpu as pltpu
from jax.experimental.pallas import tpu_sc as plsc

N = 100000
P = 102400
F = 16
BLK = 4096
GRID = P // BLK
E = 1600000
NW = 32
EPT = 50688
EPAD = NW * EPT
K = 3
B = 256
NCH = EPT // (K * B)
ROWS_PER_W = EPT // B
TR = P // 16

_mesh = plsc.VectorSubcoreMesh(core_axis_name="c", subcore_axis_name="s")
_sc_params = pltpu.CompilerParams(use_tc_tiling_on_sc=False)


@functools.partial(
    pl.kernel,
    out_type=jax.ShapeDtypeStruct((2, P), jnp.float32),
    mesh=_mesh,
    scratch_types=[
        pltpu.VMEM((K, B), jnp.int32),
        pltpu.VMEM((B,), jnp.float32),
        pltpu.VMEM((TR,), jnp.float32),
        pltpu.VMEM_SHARED((P,), jnp.float32),
    ],
    compiler_params=_sc_params,
)
def _deg_kernel(dst_r, out_r, didx, ones, zb, accd):
    c = lax.axis_index("c")
    s = lax.axis_index("s")
    r0 = s * TR

    def zfill(i, carry):
        zb[pl.ds(i * 16, 16)] = jnp.zeros((16,), jnp.float32)
        return carry
    lax.fori_loop(0, TR // 16, zfill, 0)
    for j in range(B // 16):
        ones[pl.ds(j * 16, 16)] = jnp.ones((16,), jnp.float32)
    pltpu.sync_copy(zb, accd.at[pl.ds(r0, TR)])
    plsc.subcore_barrier()

    w = c * 16 + s
    row0 = w * ROWS_PER_W

    def chunk(i, carry):
        pltpu.sync_copy(dst_r.at[pl.ds(row0 + i * K, K)], didx)
        for j in range(K):
            pltpu.sync_copy(ones, accd.at[didx.at[j]], add=True)
        return carry
    lax.fori_loop(0, NCH, chunk, 0)

    plsc.subcore_barrier()
    pltpu.sync_copy(accd.at[pl.ds(r0, TR)], out_r.at[c].at[pl.ds(r0, TR)])


@functools.partial(
    pl.kernel,
    out_type=jax.ShapeDtypeStruct((2, P, F), jnp.float32),
    mesh=_mesh,
    scratch_types=[
        pltpu.VMEM((2, K, B), jnp.int32),
        pltpu.VMEM((2, K, B), jnp.int32),
        pltpu.VMEM((2, K, B, F), jnp.float32),
        pltpu.VMEM_SHARED((P, F), jnp.float32),
        pltpu.SemaphoreType.DMA,
        pltpu.SemaphoreType.DMA,
    ],
    compiler_params=_sc_params,
)
def _edge_kernel(src_r, dst_r, hn_r, out_r, sidx, didx, rows, acc,
                 sem0, sem1):
    c = lax.axis_index("c")
    s = lax.axis_index("s")
    r0 = s * TR
    pltpu.sync_copy(hn_r.at[pl.ds(r0, TR)], acc.at[pl.ds(r0, TR)])
    plsc.subcore_barrier()

    w = c * 16 + s
    row0 = w * ROWS_PER_W
    sems = (sem0, sem1)

    def load_fire(chunk, b):
        rbase = row0 + chunk * K
        pltpu.sync_copy(src_r.at[pl.ds(rbase, K)], sidx.at[b])
        pltpu.sync_copy(dst_r.at[pl.ds(rbase, K)], didx.at[b])
        for j in range(K):
            pltpu.async_copy(hn_r.at[sidx.at[b].at[j]], rows.at[b].at[j],
                             sems[b])

    def drain_scatter(b):
        for j in range(K):
            pltpu.make_async_copy(hn_r.at[sidx.at[b].at[j]],
                                  rows.at[b].at[j], sems[b]).wait()
        for j in range(K):
            pltpu.sync_copy(rows.at[b].at[j], acc.at[didx.at[b].at[j]],
                            add=True)

    load_fire(0, 0)
    load_fire(1, 1)

    def body(t, carry):
        cbase = 2 * t
        drain_scatter(0)
        load_fire(cbase + 2, 0)
        drain_scatter(1)
        load_fire(cbase + 3, 1)
        return carry
    lax.fori_loop(0, NCH // 2 - 1, body, 0)
    drain_scatter(0)
    drain_scatter(1)

    plsc.subcore_barrier()
    pltpu.sync_copy(acc.at[pl.ds(r0, TR)], out_r.at[c].at[pl.ds(r0, TR)])


def _node_mask(i):
    rows = lax.broadcasted_iota(jnp.int32, (BLK, F), 0)
    return (i * BLK + rows) < N


def _tcA_body(x1_ref, x2_ref, W4a_ref, W4b_ref, b4_ref, WgB_ref,
              W1_ref, b1_ref, WgA_ref, c_ref, h0_ref, cs_ref):
    i = pl.program_id(0)
    x1b = x1_ref[...]
    x2b = x2_ref[...]
    bf = (jnp.dot(x1b, W4a_ref[...], preferred_element_type=jnp.float32)
          + jnp.dot(x2b, W4b_ref[...], preferred_element_type=jnp.float32)
          + b4_ref[...])
    cc = jnp.dot(bf, WgB_ref[...], preferred_element_type=jnp.float32)
    c_ref[...] = cc
    a0 = jnp.maximum(
        jnp.dot(x1b, W1_ref[...], preferred_element_type=jnp.float32)
        + b1_ref[...], 0.0)
    h0_ref[...] = jnp.dot(a0, WgA_ref[...],
                          preferred_element_type=jnp.float32) + cc

    @pl.when(i == 0)
    def _():
        cs_ref[...] = jnp.zeros_like(cs_ref)
    cs_ref[...] += jnp.sum(bf, axis=0, keepdims=True)


def _tcB_body(deg_ref, h0_ref, dinv_ref, hn_ref):
    i = pl.program_id(0)
    d = deg_ref[...]
    dn = lax.rsqrt(1.0 + d[0] + d[1])
    dinv_ref[...] = dn
    hn_ref[...] = jnp.where(_node_mask(i), h0_ref[...] * dn, 0.0)


def _tcL_body(a0_ref, a1_ref, hn_ref, dinv_ref, c_ref, WgA_ref, bg_ref,
              out_ref):
    i = pl.program_id(0)
    hn = hn_ref[...]
    sacc = a0_ref[0] + a1_ref[0] - hn
    dinv = dinv_ref[...]
    m = _node_mask(i)
    a = jnp.where(m, jnp.maximum(dinv * sacc + bg_ref[...], 0.0), 0.0)
    h = jnp.dot(a, WgA_ref[...], preferred_element_type=jnp.float32)
    out_ref[...] = jnp.where(m, (h + c_ref[...]) * dinv, 0.0)


def _tcF_body(a0_ref, a1_ref, hn_ref, dinv_ref, bg_ref, csbf_ref,
              w3a_ref, w3b_ref, b3_ref, cs_ref, res_ref):
    i = pl.program_id(0)
    hn = hn_ref[...]
    sacc = a0_ref[0] + a1_ref[0] - hn
    a = jnp.where(_node_mask(i),
                  jnp.maximum(dinv_ref[...] * sacc + bg_ref[...], 0.0), 0.0)

    @pl.when(i == 0)
    def _():
        cs_ref[...] = jnp.zeros_like(cs_ref)
    cs_ref[...] += jnp.sum(a, axis=0, keepdims=True)

    @pl.when(i == pl.num_programs(0) - 1)
    def _():
        t = (jnp.sum(cs_ref[...] * w3a_ref[...])
             + jnp.sum(csbf_ref[...] * w3b_ref[...])) / jnp.float32(N) \
            + b3_ref[0, 0]
        res_ref[...] = jnp.tanh(t) * jnp.ones((1, 1), jnp.float32)


def _row_spec():
    return pl.BlockSpec((BLK, F), lambda i: (i, 0))


def _col_spec():
    return pl.BlockSpec((BLK, 1), lambda i: (i, 0))


def _const_spec(shape):
    return pl.BlockSpec(shape, lambda i: tuple(0 for _ in shape))


_tcA = pl.pallas_call(
    _tcA_body,
    grid=(20,),
    in_specs=[pl.BlockSpec((5000, 15), lambda i: (i, 0)),
              _const_spec((5000, 4)),
              _const_spec((15, 32)), _const_spec((4, 32)),
              _const_spec((1, 32)), _const_spec((32, F)),
              _const_spec((15, F)), _const_spec((1, F)),
              _const_spec((F, F))],
    out_specs=[pl.BlockSpec((5000, F), lambda i: (i, 0)),
               pl.BlockSpec((5000, F), lambda i: (i, 0)),
               _const_spec((1, 32))],
    out_shape=[jax.ShapeDtypeStruct((N, F), jnp.float32),
               jax.ShapeDtypeStruct((N, F), jnp.float32),
               jax.ShapeDtypeStruct((1, 32), jnp.float32)],
)

_tcB = pl.pallas_call(
    _tcB_body,
    grid=(GRID,),
    in_specs=[pl.BlockSpec((2, BLK, 1), lambda i: (0, i, 0)), _row_spec()],
    out_specs=[_col_spec(), _row_spec()],
    out_shape=[jax.ShapeDtypeStruct((P, 1), jnp.float32),
               jax.ShapeDtypeStruct((P, F), jnp.float32)],
)

_acc0_spec = pl.BlockSpec((1, BLK, F), lambda i: (0, i, 0))
_acc1_spec = pl.BlockSpec((1, BLK, F), lambda i: (1, i, 0))

_tcL = pl.pallas_call(
    _tcL_body,
    grid=(GRID,),
    in_specs=[_acc0_spec, _acc1_spec, _row_spec(), _col_spec(), _row_spec(),
              _const_spec((F, F)), _const_spec((1, F))],
    out_specs=_row_spec(),
    out_shape=jax.ShapeDtypeStruct((P, F), jnp.float32),
)

_tcF = pl.pallas_call(
    _tcF_body,
    grid=(GRID,),
    in_specs=[_acc0_spec, _acc1_spec, _row_spec(), _col_spec(),
              _const_spec((1, F)), _const_spec((1, 32)),
              _const_spec((1, F)), _const_spec((1, 32)),
              _const_spec((1, 1))],
    out_specs=[_const_spec((1, F)), _const_spec((1, 1))],
    out_shape=[jax.ShapeDtypeStruct((1, F), jnp.float32),
               jax.ShapeDtypeStruct((1, 1), jnp.float32)],
)


def kernel(x1, x2, edges, W1, b1, Wg, bg, W3, b3, W4, b4):
    f32 = jnp.float32
    W4a = jnp.zeros((15, 32), f32).at[:, :19].set(W4[:15])
    W4b = jnp.zeros((4, 32), f32).at[:, :19].set(W4[15:])
    b4p = jnp.zeros((1, 32), f32).at[0, :19].set(b4)
    WgBp = jnp.zeros((32, F), f32).at[:19, :15].set(Wg[15:])
    W1p = jnp.zeros((15, F), f32).at[:, :15].set(W1)
    b1p = jnp.zeros((1, F), f32).at[0, :15].set(b1)
    WgAp = jnp.zeros((F, F), f32).at[:15, :15].set(Wg[:15])
    bg16 = jnp.zeros((1, F), f32).at[0, :15].set(bg)
    w3a = jnp.zeros((1, F), f32).at[0, :15].set(W3[:15, 0])
    w3b = jnp.zeros((1, 32), f32).at[0, :19].set(W3[15:, 0])
    b3s = jnp.reshape(b3, (1, 1)).astype(f32)

    npad = EPAD - E
    padi = (N + (jnp.arange(npad, dtype=jnp.int32) % 64))
    src2d = jnp.concatenate([edges[0], padi]).reshape(EPAD // B, B)
    dst2d = jnp.concatenate([edges[1], padi]).reshape(EPAD // B, B)

    deg = _deg_kernel(dst2d)
    deg3 = jnp.reshape(deg, (2, P, 1))
    c, h0, csum_bf = _tcA(x1, x2, W4a, W4b, b4p, WgBp, W1p, b1p, WgAp)
    cp = jnp.pad(c, ((0, P - N), (0, 0)))
    h0p = jnp.pad(h0, ((0, P - N), (0, 0)))
    dinv, hn = _tcB(deg3, h0p)
    for k in range(5):
        acc = _edge_kernel(src2d, dst2d, hn)
        if k < 4:
            hn = _tcL(acc, acc, hn, dinv, cp, WgAp, bg16)
        else:
            _, res = _tcF(acc, acc, hn, dinv, bg16, csum_bf, w3a, w3b, b3s)
    return jnp.reshape(res, ())
